# baseline (speedup 1.0000x reference)
import jax
import jax.numpy as jnp
from jax import lax
from jax.experimental import pallas as pl
from jax.experimental.pallas import tpu as pltpu
from jax.experimental.pallas import tpu_sc as plsc

VOCAB = 1000000
VEC = 64
SEQ = 50
BATCH = 4096

NC = 2
NS = 16
NW = NC * NS
BPW = BATCH // NW
CHUNK = 16
NCHUNK = BPW // CHUNK
ROWS = CHUNK * SEQ
NV = VEC // 16


def _cbow_body(text_flat, w_vec, b_vec, table, out_hbm,
               idx_v, rows0, rows1, w_v, b_v, out_v, sem0, sem1):
  cid = lax.axis_index("c")
  sid = lax.axis_index("s")
  wid = sid * NC + cid
  base = wid * BPW

  pltpu.sync_copy(text_flat.at[pl.ds(base * SEQ, BPW * SEQ)], idx_v)
  pltpu.sync_copy(w_vec, w_v)
  pltpu.sync_copy(b_vec, b_v)

  w_regs = [w_v[pl.ds(k * 16, 16)] for k in range(NV)]
  bias_v = b_v[...]
  lane = lax.iota(jnp.int32, 16)

  def hsum(x):
    for sh in (8, 4, 2, 1):
      x = x + x.at[lane ^ sh].get(mode="promise_in_bounds")
    return x

  def start(ci, buf, sem):
    pltpu.async_copy(table.at[idx_v.at[pl.ds(ci * ROWS, ROWS)]], buf, sem)

  def wait(buf, sem):
    pltpu.make_async_copy(table.at[pl.ds(0, ROWS)], buf, sem).wait()

  def compute(ci, buf):
    def body(c, ovec):
      r0 = c * SEQ
      accs = [buf[r0, pl.ds(k * 16, 16)] for k in range(NV)]
      for s in range(1, SEQ):
        for k in range(NV):
          accs[k] = accs[k] + buf[r0 + s, pl.ds(k * 16, 16)]
      p = jnp.maximum(accs[0], 0.0) * w_regs[0]
      for k in range(1, NV):
        p = p + jnp.maximum(accs[k], 0.0) * w_regs[k]
      total = hsum(p) + bias_v
      return jnp.where(lane == c, total, ovec)
    ovec = lax.fori_loop(0, CHUNK, body, jnp.zeros((16,), jnp.float32))
    out_v[pl.ds(ci * CHUNK, CHUNK)] = ovec

  start(0, rows0, sem0)

  def outer(gg, carry):
    start(2 * gg + 1, rows1, sem1)
    wait(rows0, sem0)
    compute(2 * gg, rows0)

    @pl.when(gg < NCHUNK // 2 - 1)
    def _():
      start(2 * gg + 2, rows0, sem0)

    wait(rows1, sem1)
    compute(2 * gg + 1, rows1)
    return carry

  lax.fori_loop(0, NCHUNK // 2, outer, 0)

  pltpu.sync_copy(out_v, out_hbm.at[pl.ds(base, BPW)])


def kernel(text, W, w_lin, b_lin):
  text_flat = text.T.reshape(-1)
  w64 = w_lin.reshape(VEC)
  b16 = jnp.broadcast_to(b_lin, (16,))

  mesh = plsc.VectorSubcoreMesh(core_axis_name="c", subcore_axis_name="s")
  kern = pl.kernel(
      _cbow_body,
      mesh=mesh,
      compiler_params=pltpu.CompilerParams(use_tc_tiling_on_sc=False),
      out_type=jax.ShapeDtypeStruct((BATCH,), jnp.float32),
      scratch_types=[
          pltpu.VMEM((BPW * SEQ,), jnp.int32),
          pltpu.VMEM((ROWS, VEC), jnp.float32),
          pltpu.VMEM((ROWS, VEC), jnp.float32),
          pltpu.VMEM((VEC,), jnp.float32),
          pltpu.VMEM((16,), jnp.float32),
          pltpu.VMEM((BPW,), jnp.float32),
          pltpu.SemaphoreType.DMA,
          pltpu.SemaphoreType.DMA,
      ],
  )
  return kern(text_flat, w64, b16, W)


# speedup vs baseline: 1.0112x; 1.0112x over previous
import jax
import jax.numpy as jnp
from jax import lax
from jax.experimental import pallas as pl
from jax.experimental.pallas import tpu as pltpu
from jax.experimental.pallas import tpu_sc as plsc

VOCAB = 1000000
VEC = 64
SEQ = 50
BATCH = 4096

NC = 2
NS = 16
NW = NC * NS
BPW = BATCH // NW
SCH = 5
NCHUNK = SEQ // SCH
NV = VEC // 16


def _cbow_body(text, w_vec, b_vec, table, out_hbm,
               idx_v, buf0, buf1, acc_v, w_v, b_v, out_v, sem0, sem1):
  cid = lax.axis_index("c")
  sid = lax.axis_index("s")
  wid = sid * NC + cid
  base = wid * BPW

  pltpu.sync_copy(text.at[:, pl.ds(base, BPW)], idx_v)
  pltpu.sync_copy(w_vec, w_v)
  pltpu.sync_copy(b_vec, b_v)

  w_regs = [w_v[pl.ds(k * 16, 16)] for k in range(NV)]
  bias_v = b_v[...]
  lane = lax.iota(jnp.int32, 16)
  zero = jnp.zeros((16,), jnp.float32)

  def hsum(x):
    for sh in (8, 4, 2, 1):
      x = x + x.at[lane ^ sh].get(mode="promise_in_bounds")
    return x

  def zbody(c, carry):
    for k in range(NV):
      acc_v[c, pl.ds(k * 16, 16)] = zero
    return carry

  lax.fori_loop(0, BPW, zbody, 0)

  def start(ci, buf, sem):
    for j in range(SCH):
      pltpu.async_copy(table.at[idx_v.at[ci * SCH + j]], buf.at[j], sem)

  def wait(buf, sem):
    for s in range(SCH):
      pltpu.make_async_copy(table.at[pl.ds(0, BPW)], buf.at[s], sem).wait()

  def accumulate(buf):
    def body(c, carry):
      for k in range(NV):
        v = buf[0, c, pl.ds(k * 16, 16)]
        for s in range(1, SCH):
          v = v + buf[s, c, pl.ds(k * 16, 16)]
        plsc.addupdate(acc_v.at[c, pl.ds(k * 16, 16)], v)
      return carry
    lax.fori_loop(0, BPW, body, 0)

  start(0, buf0, sem0)

  def outer(gg, carry):
    start(2 * gg + 1, buf1, sem1)
    wait(buf0, sem0)
    accumulate(buf0)

    @pl.when(gg < NCHUNK // 2 - 1)
    def _():
      start(2 * gg + 2, buf0, sem0)

    wait(buf1, sem1)
    accumulate(buf1)
    return carry

  lax.fori_loop(0, NCHUNK // 2, outer, 0)

  def fgroup(g, carry):
    ovec = zero
    for j in range(16):
      c = g * 16 + j
      accs = [acc_v[c, pl.ds(k * 16, 16)] for k in range(NV)]
      p = jnp.maximum(accs[0], 0.0) * w_regs[0]
      for k in range(1, NV):
        p = p + jnp.maximum(accs[k], 0.0) * w_regs[k]
      total = hsum(p) + bias_v
      ovec = jnp.where(lane == j, total, ovec)
    out_v[pl.ds(g * 16, 16)] = ovec
    return carry

  lax.fori_loop(0, BPW // 16, fgroup, 0)

  pltpu.sync_copy(out_v, out_hbm.at[pl.ds(base, BPW)])


def kernel(text, W, w_lin, b_lin):
  w64 = w_lin.reshape(VEC)
  b16 = jnp.broadcast_to(b_lin, (16,))

  mesh = plsc.VectorSubcoreMesh(core_axis_name="c", subcore_axis_name="s")
  kern = pl.kernel(
      _cbow_body,
      mesh=mesh,
      compiler_params=pltpu.CompilerParams(use_tc_tiling_on_sc=False),
      out_type=jax.ShapeDtypeStruct((BATCH,), jnp.float32),
      scratch_types=[
          pltpu.VMEM((SEQ, BPW), jnp.int32),
          pltpu.VMEM((SCH, BPW, VEC), jnp.float32),
          pltpu.VMEM((SCH, BPW, VEC), jnp.float32),
          pltpu.VMEM((BPW, VEC), jnp.float32),
          pltpu.VMEM((VEC,), jnp.float32),
          pltpu.VMEM((16,), jnp.float32),
          pltpu.VMEM((BPW,), jnp.float32),
          pltpu.SemaphoreType.DMA,
          pltpu.SemaphoreType.DMA,
      ],
  )
  return kern(text, w64, b16, W)
